# baseline (speedup 1.0000x reference)
import functools

import jax
import jax.numpy as jnp
from jax.experimental import pallas as pl
from jax.experimental.pallas import tpu as pltpu

_LN_EPS = 1e-5


def _ln_feature_major(h, inv_n):
    mu = jnp.sum(h, axis=0, keepdims=True) * inv_n
    ex2 = jnp.sum(h * h, axis=0, keepdims=True) * inv_n
    var = jnp.maximum(ex2 - mu * mu, 0.0)
    return (h - mu) * jax.lax.rsqrt(var + _LN_EPS)


def _actor_body(x_ref, w1_ref, b1_ref, w2_ref, b2_ref, w3_ref, b3_ref, o_ref,
                *, hidden):
    inv_h = 1.0 / float(hidden)
    w1 = w1_ref[...]
    w2 = w2_ref[...]
    w3 = w3_ref[...]
    b1 = b1_ref[...]
    b2 = b2_ref[...]
    b3 = b3_ref[...]

    def lnt_sliced(h, c=128):
        n = h.shape[1]
        if n <= c:
            return jnp.tanh(_ln_feature_major(h, inv_h)).astype(jnp.bfloat16)
        return jnp.concatenate(
            [jnp.tanh(_ln_feature_major(h[:, s:s + c], inv_h)
                      ).astype(jnp.bfloat16) for s in range(0, n, c)], axis=1)

    x = x_ref[...].astype(jnp.bfloat16)
    h1 = jax.lax.dot_general(
        w1, x, (((1,), (1,)), ((), ())),
        preferred_element_type=jnp.float32) + b1
    a1 = lnt_sliced(h1)
    h2 = jax.lax.dot_general(
        w2, a1, (((1,), (0,)), ((), ())),
        preferred_element_type=jnp.float32) + b2
    a2 = lnt_sliced(h2)
    bt = a2.shape[1]
    oc = 1024 if bt % 1024 == 0 else bt
    for s in range(0, bt, oc):
        h3 = jax.lax.dot_general(
            a2[:, s:s + oc], w3, (((0,), (1,)), ((), ())),
            preferred_element_type=jnp.float32) + b3
        o_ref[s:s + oc, :] = jnp.tanh(h3)


def kernel(x, w1, b1, w2, b2, w3, b3):
    batch, input_dim = x.shape
    hidden = w1.shape[0]
    output_dim = w3.shape[0]
    batch_tile = 4096

    padded = ((batch + batch_tile - 1) // batch_tile) * batch_tile
    if padded != batch:
        x = jnp.pad(x, ((0, padded - batch), (0, 0)))
    n_tiles = padded // batch_tile

    w1b = w1.astype(jnp.bfloat16)
    w2b = w2.astype(jnp.bfloat16)
    w3b = w3.astype(jnp.bfloat16)
    b1c = b1.reshape(hidden, 1).astype(jnp.float32)
    b2c = b2.reshape(hidden, 1).astype(jnp.float32)
    b3r = b3.reshape(1, output_dim).astype(jnp.float32)

    flops = 2 * padded * (input_dim * hidden + hidden * hidden
                          + hidden * output_dim)
    trans = padded * (2 * hidden + output_dim + 2)
    bytes_accessed = 4 * padded * (input_dim + output_dim) + 2 * (
        input_dim * hidden + hidden * hidden + hidden * output_dim)

    def resident(shape):
        return pl.BlockSpec(shape, lambda *_: (0,) * len(shape))

    body = functools.partial(_actor_body, hidden=hidden)
    out = pl.pallas_call(
        body,
        out_shape=jax.ShapeDtypeStruct((padded, output_dim), jnp.float32),
        grid=(n_tiles,),
        in_specs=[
            pl.BlockSpec((batch_tile, input_dim), lambda i: (i, 0)),
            resident((hidden, input_dim)),
            resident((hidden, 1)),
            resident((hidden, hidden)),
            resident((hidden, 1)),
            resident((output_dim, hidden)),
            resident((1, output_dim)),
        ],
        out_specs=pl.BlockSpec((batch_tile, output_dim), lambda i: (i, 0)),
        compiler_params=pltpu.CompilerParams(
            dimension_semantics=("parallel",)),
        cost_estimate=pl.CostEstimate(flops=int(flops),
                                      transcendentals=int(trans),
                                      bytes_accessed=int(bytes_accessed)),
    )(x, w1b, b1c, w2b, b2c, w3b, b3r)
    return out[:batch]


# speedup vs baseline: 1.0057x; 1.0057x over previous
import functools

import jax
import jax.numpy as jnp
from jax.experimental import pallas as pl
from jax.experimental.pallas import tpu as pltpu

_LN_EPS = 1e-5


def _ln_feature_major(h, inv_n):
    mu = jnp.sum(h, axis=0, keepdims=True) * inv_n
    ex2 = jnp.sum(h * h, axis=0, keepdims=True) * inv_n
    var = jnp.maximum(ex2 - mu * mu, 0.0)
    return (h - mu) * jax.lax.rsqrt(var + _LN_EPS)


def _actor_body(x_ref, w1_ref, b1_ref, w2_ref, b2_ref, w3_ref, b3_ref, o_ref,
                *, hidden):
    inv_h = 1.0 / float(hidden)
    w1 = w1_ref[...]
    w2 = w2_ref[...]
    w3 = w3_ref[...]
    b1 = b1_ref[...]
    b2 = b2_ref[...]
    b3 = b3_ref[...]

    def lnt_sliced(h, c=128):
        n = h.shape[1]
        if n <= c:
            return jnp.tanh(_ln_feature_major(h, inv_h)).astype(jnp.bfloat16)
        return jnp.concatenate(
            [jnp.tanh(_ln_feature_major(h[:, s:s + c], inv_h)
                      ).astype(jnp.bfloat16) for s in range(0, n, c)], axis=1)

    x = x_ref[...].astype(jnp.bfloat16)
    h1 = jax.lax.dot_general(
        w1, x, (((1,), (1,)), ((), ())),
        preferred_element_type=jnp.float32) + b1
    a1 = lnt_sliced(h1)
    h2 = jax.lax.dot_general(
        w2, a1, (((1,), (0,)), ((), ())),
        preferred_element_type=jnp.float32) + b2
    a2 = lnt_sliced(h2)
    h3 = jax.lax.dot_general(
        a2, w3, (((0,), (1,)), ((), ())),
        preferred_element_type=jnp.float32) + b3
    o_ref[...] = jnp.tanh(h3)


def kernel(x, w1, b1, w2, b2, w3, b3):
    batch, input_dim = x.shape
    hidden = w1.shape[0]
    output_dim = w3.shape[0]
    batch_tile = 4096

    padded = ((batch + batch_tile - 1) // batch_tile) * batch_tile
    if padded != batch:
        x = jnp.pad(x, ((0, padded - batch), (0, 0)))
    n_tiles = padded // batch_tile

    w1b = w1.astype(jnp.bfloat16)
    w2b = w2.astype(jnp.bfloat16)
    w3b = w3.astype(jnp.bfloat16)
    b1c = b1.reshape(hidden, 1).astype(jnp.float32)
    b2c = b2.reshape(hidden, 1).astype(jnp.float32)
    b3r = b3.reshape(1, output_dim).astype(jnp.float32)

    flops = 2 * padded * (input_dim * hidden + hidden * hidden
                          + hidden * output_dim)
    trans = padded * (2 * hidden + output_dim + 2)
    bytes_accessed = 4 * padded * (input_dim + output_dim) + 2 * (
        input_dim * hidden + hidden * hidden + hidden * output_dim)

    def resident(shape):
        return pl.BlockSpec(shape, lambda *_: (0,) * len(shape))

    body = functools.partial(_actor_body, hidden=hidden)
    out = pl.pallas_call(
        body,
        out_shape=jax.ShapeDtypeStruct((padded, output_dim), jnp.float32),
        grid=(n_tiles,),
        in_specs=[
            pl.BlockSpec((batch_tile, input_dim), lambda i: (i, 0)),
            resident((hidden, input_dim)),
            resident((hidden, 1)),
            resident((hidden, hidden)),
            resident((hidden, 1)),
            resident((output_dim, hidden)),
            resident((1, output_dim)),
        ],
        out_specs=pl.BlockSpec((batch_tile, output_dim), lambda i: (i, 0)),
        compiler_params=pltpu.CompilerParams(
            dimension_semantics=("parallel",)),
        cost_estimate=pl.CostEstimate(flops=int(flops),
                                      transcendentals=int(trans),
                                      bytes_accessed=int(bytes_accessed)),
    )(x, w1b, b1c, w2b, b2c, w3b, b3r)
    return out[:batch]
